# baseline (speedup 1.0000x reference)
import jax
import jax.numpy as jnp
from jax import lax
from jax.experimental import pallas as pl
from jax.experimental.pallas import tpu as pltpu
from jax.experimental.pallas import tpu_sc as plsc

N = 10000
F = 256
H = 512
C = 128
E = 160000

NC = 2
NS = 16
B = 128

T = 10240
RPT = T // NS

CPT_D = 40
CPT_B = 80
EPAD = NC * NS * CPT_D * B
ECH = EPAD // B

_mesh = plsc.VectorSubcoreMesh(core_axis_name="c", subcore_axis_name="s")


def _deg_body(dst_hbm, ones_hbm, z_hbm, out_hbm, dst_i, ones_v, acc_sh, sem):
    c = lax.axis_index("c")
    s = lax.axis_index("s")
    w = c * NS + s
    pltpu.sync_copy(dst_hbm.at[pl.ds(w * CPT_D, CPT_D)], dst_i)
    pltpu.sync_copy(z_hbm.at[pl.ds(s * RPT, RPT)],
                    acc_sh.at[pl.ds(s * RPT, RPT)])
    pltpu.sync_copy(ones_hbm, ones_v)
    plsc.subcore_barrier()

    def chunk(g, carry):
        pltpu.sync_copy(ones_v, acc_sh.at[dst_i.at[g]], add=True)
        return carry

    lax.fori_loop(0, CPT_D, chunk, 0)
    plsc.subcore_barrier()
    pltpu.sync_copy(acc_sh.at[pl.ds(s * RPT, RPT)],
                    out_hbm.at[c, pl.ds(s * RPT, RPT)])


_sc_deg = pl.kernel(
    _deg_body,
    out_type=jax.ShapeDtypeStruct((NC, T, C), jnp.float32),
    scratch_types=[
        pltpu.VMEM((CPT_D, B), jnp.int32),
        pltpu.VMEM((B, C), jnp.float32),
        pltpu.VMEM_SHARED((T, C), jnp.float32),
        pltpu.SemaphoreType.DMA,
    ],
    mesh=_mesh,
)


def _agg_pipeline(table_hbm, src_i, dst_i, rows0, rows1, acc_sh,
                  sem0, sem1, cpt):
    pltpu.async_copy(table_hbm.at[src_i.at[0]], rows0, sem0)

    def pair(i, carry):
        g0 = 2 * i
        pltpu.async_copy(table_hbm.at[src_i.at[g0 + 1]], rows1, sem1)
        pltpu.make_async_copy(table_hbm.at[src_i.at[g0]], rows0, sem0).wait()
        pltpu.sync_copy(rows0, acc_sh.at[dst_i.at[g0]], add=True)

        @pl.when(i < cpt // 2 - 1)
        def _():
            pltpu.async_copy(table_hbm.at[src_i.at[g0 + 2]], rows0, sem0)

        pltpu.make_async_copy(table_hbm.at[src_i.at[g0 + 1]], rows1,
                              sem1).wait()
        pltpu.sync_copy(rows1, acc_sh.at[dst_i.at[g0 + 1]], add=True)
        return carry

    lax.fori_loop(0, cpt // 2, pair, 0)


def _aggB_body(xs_hbm, srcs_hbm, dsts_hbm, out_hbm,
               src_i, dst_i, rows0, rows1, acc_sh, sem0, sem1):
    c = lax.axis_index("c")
    s = lax.axis_index("s")
    pltpu.sync_copy(xs_hbm.at[pl.ds(c * T + s * RPT, RPT)],
                    acc_sh.at[pl.ds(s * RPT, RPT)])
    plsc.subcore_barrier()
    for p in range(CPT_B // CPT_D):
        pltpu.sync_copy(
            srcs_hbm.at[pl.ds(c * (NS * CPT_B) + s * CPT_B + p * CPT_D,
                              CPT_D)], src_i)
        pltpu.sync_copy(dsts_hbm.at[pl.ds(s * CPT_B + p * CPT_D, CPT_D)],
                        dst_i)
        _agg_pipeline(xs_hbm, src_i, dst_i, rows0, rows1, acc_sh,
                      sem0, sem1, CPT_D)
    plsc.subcore_barrier()
    pltpu.sync_copy(acc_sh.at[pl.ds(s * RPT, RPT)],
                    out_hbm.at[c, pl.ds(s * RPT, RPT)])


_sc_aggB = pl.kernel(
    _aggB_body,
    out_type=jax.ShapeDtypeStruct((NC, T, C), jnp.float32),
    scratch_types=[
        pltpu.VMEM((CPT_D, B), jnp.int32),
        pltpu.VMEM((CPT_D, B), jnp.int32),
        pltpu.VMEM((B, C), jnp.float32),
        pltpu.VMEM((B, C), jnp.float32),
        pltpu.VMEM_SHARED((T, C), jnp.float32),
        pltpu.SemaphoreType.DMA,
        pltpu.SemaphoreType.DMA,
    ],
    mesh=_mesh,
)


def _aggD_body(ts_hbm, srcs_hbm, dsts_hbm, z_hbm, out_hbm,
               src_i, dst_i, rows0, rows1, acc_sh, sem0, sem1):
    c = lax.axis_index("c")
    s = lax.axis_index("s")
    w = c * NS + s
    pltpu.sync_copy(srcs_hbm.at[pl.ds(w * CPT_D, CPT_D)], src_i)
    pltpu.sync_copy(dsts_hbm.at[pl.ds(w * CPT_D, CPT_D)], dst_i)

    @pl.when(c == 0)
    def _():
        pltpu.sync_copy(ts_hbm.at[pl.ds(s * RPT, RPT)],
                        acc_sh.at[pl.ds(s * RPT, RPT)])

    @pl.when(c == 1)
    def _():
        pltpu.sync_copy(z_hbm.at[pl.ds(s * RPT, RPT)],
                        acc_sh.at[pl.ds(s * RPT, RPT)])

    plsc.subcore_barrier()
    _agg_pipeline(ts_hbm, src_i, dst_i, rows0, rows1, acc_sh,
                  sem0, sem1, CPT_D)
    plsc.subcore_barrier()
    pltpu.sync_copy(acc_sh.at[pl.ds(s * RPT, RPT)],
                    out_hbm.at[c, pl.ds(s * RPT, RPT)])


_sc_aggD = pl.kernel(
    _aggD_body,
    out_type=jax.ShapeDtypeStruct((NC, T, C), jnp.float32),
    scratch_types=[
        pltpu.VMEM((CPT_D, B), jnp.int32),
        pltpu.VMEM((CPT_D, B), jnp.int32),
        pltpu.VMEM((B, C), jnp.float32),
        pltpu.VMEM((B, C), jnp.float32),
        pltpu.VMEM_SHARED((T, C), jnp.float32),
        pltpu.SemaphoreType.DMA,
        pltpu.SemaphoreType.DMA,
    ],
    mesh=_mesh,
)


RT = T // NS
R = 400
GRID = N // R


def _tcA_body(d0_ref, d1_ref, x_ref, xs_ref, dis_ref):
    deg = d0_ref[:, 0:1] + d1_ref[:, 0:1] + 1.0
    dis = lax.rsqrt(deg)
    xs_ref[...] = x_ref[...] * dis
    dis_ref[...] = dis


_tc_a = pl.pallas_call(
    _tcA_body,
    grid=(NC, NS),
    in_specs=[
        pl.BlockSpec((RT, C), lambda h, i: (i, 0)),
        pl.BlockSpec((RT, C), lambda h, i: (i, 0)),
        pl.BlockSpec((RT, C), lambda h, i: (i, h)),
    ],
    out_specs=[
        pl.BlockSpec((RT, C), lambda h, i: (h * NS + i, 0)),
        pl.BlockSpec((RT, 1), lambda h, i: (i, 0)),
    ],
    out_shape=[
        jax.ShapeDtypeStruct((2 * T, C), jnp.float32),
        jax.ShapeDtypeStruct((T, 1), jnp.float32),
    ],
)


def _tcC_body(a0_ref, a1_ref, dis_ref, w1a_ref, w1b_ref, b1_ref, w2_ref,
              ts_ref):
    dis = dis_ref[...]
    h = (jnp.dot(a0_ref[...] * dis, w1a_ref[...],
                 preferred_element_type=jnp.float32)
         + jnp.dot(a1_ref[...] * dis, w1b_ref[...],
                   preferred_element_type=jnp.float32)
         + b1_ref[...])
    h = jnp.maximum(h, 0.0)
    t = jnp.dot(h, w2_ref[...], preferred_element_type=jnp.float32)
    ts_ref[...] = t * dis


_tc_c = pl.pallas_call(
    _tcC_body,
    grid=(NS,),
    in_specs=[
        pl.BlockSpec((RT, C), lambda i: (i, 0)),
        pl.BlockSpec((RT, C), lambda i: (i, 0)),
        pl.BlockSpec((RT, 1), lambda i: (i, 0)),
        pl.BlockSpec((C, H), lambda i: (0, 0)),
        pl.BlockSpec((C, H), lambda i: (0, 0)),
        pl.BlockSpec((1, H), lambda i: (0, 0)),
        pl.BlockSpec((H, C), lambda i: (0, 0)),
    ],
    out_specs=pl.BlockSpec((RT, C), lambda i: (i, 0)),
    out_shape=jax.ShapeDtypeStruct((T, C), jnp.float32),
)


def _tcE_body(a0_ref, a1_ref, dis_ref, b2_ref, out_ref):
    out_ref[...] = ((a0_ref[...] + a1_ref[...]) * dis_ref[...]
                    + b2_ref[...])


_tc_e = pl.pallas_call(
    _tcE_body,
    grid=(GRID,),
    in_specs=[
        pl.BlockSpec((R, C), lambda i: (i, 0)),
        pl.BlockSpec((R, C), lambda i: (i, 0)),
        pl.BlockSpec((R, 1), lambda i: (i, 0)),
        pl.BlockSpec((1, C), lambda i: (0, 0)),
    ],
    out_specs=pl.BlockSpec((R, C), lambda i: (i, 0)),
    out_shape=jax.ShapeDtypeStruct((N, C), jnp.float32),
)


def kernel(x, edge_index, W1, b1, W2, b2):
    ei = edge_index.astype(jnp.int32)
    src, dst = ei[0], ei[1]

    pad = jnp.full((EPAD - E,), N, jnp.int32)
    src_d = jnp.concatenate([src, pad]).reshape(ECH, B)
    dst_d = jnp.concatenate([dst, pad]).reshape(ECH, B)

    srcs_b = jnp.concatenate([src, pad, src + T, pad + T]).reshape(2 * ECH, B)

    ones128 = jnp.ones((B, C), jnp.float32)
    zbig = jnp.zeros((T, C), jnp.float32)

    degacc = _sc_deg(dst_d, ones128, zbig)
    xs, dis = _tc_a(degacc[0], degacc[1], x)
    acc1 = _sc_aggB(xs, srcs_b, dst_d)
    ts = _tc_c(acc1[0], acc1[1], dis, W1[:C], W1[C:], b1.reshape(1, H), W2)
    acc2 = _sc_aggD(ts, src_d, dst_d, zbig)
    return _tc_e(acc2[0], acc2[1], dis, b2.reshape(1, C))


# speedup vs baseline: 1.0471x; 1.0471x over previous
import jax
import jax.numpy as jnp
from jax import lax
from jax.experimental import pallas as pl
from jax.experimental.pallas import tpu as pltpu
from jax.experimental.pallas import tpu_sc as plsc

N = 10000
F = 256
H = 512
C = 128
E = 160000

NC = 2
NS = 16
B = 128

T = 10240
RPT = T // NS

CPT_D = 40
CPT_B = 80
EPAD = NC * NS * CPT_D * B
ECH = EPAD // B

_mesh = plsc.VectorSubcoreMesh(core_axis_name="c", subcore_axis_name="s")


def _deg_body(dst_hbm, ones_hbm, z_hbm, out_hbm, dst_i, ones_v, acc_sh, sem):
    c = lax.axis_index("c")
    s = lax.axis_index("s")
    w = c * NS + s
    pltpu.sync_copy(dst_hbm.at[pl.ds(w * CPT_D, CPT_D)], dst_i)
    pltpu.sync_copy(z_hbm.at[pl.ds(s * RPT, RPT)],
                    acc_sh.at[pl.ds(s * RPT, RPT)])
    pltpu.sync_copy(ones_hbm, ones_v)
    plsc.subcore_barrier()

    def chunk(g, carry):
        pltpu.sync_copy(ones_v, acc_sh.at[dst_i.at[g]], add=True)
        return carry

    lax.fori_loop(0, CPT_D, chunk, 0)
    plsc.subcore_barrier()
    pltpu.sync_copy(acc_sh.at[pl.ds(s * RPT, RPT)],
                    out_hbm.at[c, pl.ds(s * RPT, RPT)])


_sc_deg = pl.kernel(
    _deg_body,
    out_type=jax.ShapeDtypeStruct((NC, T, C), jnp.float32),
    scratch_types=[
        pltpu.VMEM((CPT_D, B), jnp.int32),
        pltpu.VMEM((B, C), jnp.float32),
        pltpu.VMEM_SHARED((T, C), jnp.float32),
        pltpu.SemaphoreType.DMA,
    ],
    mesh=_mesh,
)


def _agg_pipeline(table_hbm, src_i, dst_i, rows0, rows1, acc_sh,
                  sem0, sem1, cpt):
    pltpu.async_copy(table_hbm.at[src_i.at[0]], rows0, sem0)

    def pair(i, carry):
        g0 = 2 * i
        pltpu.async_copy(table_hbm.at[src_i.at[g0 + 1]], rows1, sem1)
        pltpu.make_async_copy(table_hbm.at[src_i.at[g0]], rows0, sem0).wait()
        pltpu.sync_copy(rows0, acc_sh.at[dst_i.at[g0]], add=True)

        @pl.when(i < cpt // 2 - 1)
        def _():
            pltpu.async_copy(table_hbm.at[src_i.at[g0 + 2]], rows0, sem0)

        pltpu.make_async_copy(table_hbm.at[src_i.at[g0 + 1]], rows1,
                              sem1).wait()
        pltpu.sync_copy(rows1, acc_sh.at[dst_i.at[g0 + 1]], add=True)
        return carry

    lax.fori_loop(0, cpt // 2, pair, 0)


def _aggB_body(xs_hbm, srcs_hbm, dsts_hbm, out_hbm,
               src_i, dst_i, rows0, rows1, acc_sh, sem0, sem1):
    c = lax.axis_index("c")
    s = lax.axis_index("s")
    pltpu.sync_copy(xs_hbm.at[pl.ds(c * T + s * RPT, RPT)],
                    acc_sh.at[pl.ds(s * RPT, RPT)])
    plsc.subcore_barrier()
    for p in range(CPT_B // CPT_D):
        pltpu.sync_copy(
            srcs_hbm.at[pl.ds(c * (NS * CPT_B) + s * CPT_B + p * CPT_D,
                              CPT_D)], src_i)
        pltpu.sync_copy(dsts_hbm.at[pl.ds(s * CPT_B + p * CPT_D, CPT_D)],
                        dst_i)
        _agg_pipeline(xs_hbm, src_i, dst_i, rows0, rows1, acc_sh,
                      sem0, sem1, CPT_D)
    plsc.subcore_barrier()
    pltpu.sync_copy(acc_sh.at[pl.ds(s * RPT, RPT)],
                    out_hbm.at[c, pl.ds(s * RPT, RPT)])


_sc_aggB = pl.kernel(
    _aggB_body,
    out_type=jax.ShapeDtypeStruct((NC, T, C), jnp.float32),
    scratch_types=[
        pltpu.VMEM((CPT_D, B), jnp.int32),
        pltpu.VMEM((CPT_D, B), jnp.int32),
        pltpu.VMEM((B, C), jnp.float32),
        pltpu.VMEM((B, C), jnp.float32),
        pltpu.VMEM_SHARED((T, C), jnp.float32),
        pltpu.SemaphoreType.DMA,
        pltpu.SemaphoreType.DMA,
    ],
    mesh=_mesh,
)


def _aggD_body(ts_hbm, srcs_hbm, dsts_hbm, z_hbm, out_hbm,
               src_i, dst_i, rows0, rows1, acc_sh, sem0, sem1):
    c = lax.axis_index("c")
    s = lax.axis_index("s")
    w = c * NS + s
    pltpu.sync_copy(srcs_hbm.at[pl.ds(w * CPT_D, CPT_D)], src_i)
    pltpu.sync_copy(dsts_hbm.at[pl.ds(w * CPT_D, CPT_D)], dst_i)

    @pl.when(c == 0)
    def _():
        pltpu.sync_copy(ts_hbm.at[pl.ds(s * RPT, RPT)],
                        acc_sh.at[pl.ds(s * RPT, RPT)])

    @pl.when(c == 1)
    def _():
        pltpu.sync_copy(z_hbm.at[pl.ds(s * RPT, RPT)],
                        acc_sh.at[pl.ds(s * RPT, RPT)])

    plsc.subcore_barrier()
    _agg_pipeline(ts_hbm, src_i, dst_i, rows0, rows1, acc_sh,
                  sem0, sem1, CPT_D)
    plsc.subcore_barrier()
    pltpu.sync_copy(acc_sh.at[pl.ds(s * RPT, RPT)],
                    out_hbm.at[c, pl.ds(s * RPT, RPT)])


_sc_aggD = pl.kernel(
    _aggD_body,
    out_type=jax.ShapeDtypeStruct((NC, T, C), jnp.float32),
    scratch_types=[
        pltpu.VMEM((CPT_D, B), jnp.int32),
        pltpu.VMEM((CPT_D, B), jnp.int32),
        pltpu.VMEM((B, C), jnp.float32),
        pltpu.VMEM((B, C), jnp.float32),
        pltpu.VMEM_SHARED((T, C), jnp.float32),
        pltpu.SemaphoreType.DMA,
        pltpu.SemaphoreType.DMA,
    ],
    mesh=_mesh,
)


R = 400
GRID = N // R
RT = R


def _tcA_body(d0_ref, d1_ref, x_ref, xs0_ref, xs1_ref, dis_ref):
    deg = d0_ref[:, 0:1] + d1_ref[:, 0:1] + 1.0
    dis = lax.rsqrt(deg)
    xs = x_ref[...] * dis
    xs0_ref[...] = xs[:, :C]
    xs1_ref[...] = xs[:, C:]
    dis_ref[...] = dis


_tc_a = pl.pallas_call(
    _tcA_body,
    grid=(GRID,),
    in_specs=[
        pl.BlockSpec((R, C), lambda i: (i, 0)),
        pl.BlockSpec((R, C), lambda i: (i, 0)),
        pl.BlockSpec((R, F), lambda i: (i, 0)),
    ],
    out_specs=[
        pl.BlockSpec((R, C), lambda i: (i, 0)),
        pl.BlockSpec((R, C), lambda i: (i, 0)),
        pl.BlockSpec((R, 1), lambda i: (i, 0)),
    ],
    out_shape=[
        jax.ShapeDtypeStruct((N, C), jnp.float32),
        jax.ShapeDtypeStruct((N, C), jnp.float32),
        jax.ShapeDtypeStruct((N, 1), jnp.float32),
    ],
)


def _tcC_body(a0_ref, a1_ref, dis_ref, w1a_ref, w1b_ref, b1_ref, w2_ref,
              ts_ref):
    dis = dis_ref[...]
    h = (jnp.dot(a0_ref[...] * dis, w1a_ref[...],
                 preferred_element_type=jnp.float32)
         + jnp.dot(a1_ref[...] * dis, w1b_ref[...],
                   preferred_element_type=jnp.float32)
         + b1_ref[...])
    h = jnp.maximum(h, 0.0)
    t = jnp.dot(h, w2_ref[...], preferred_element_type=jnp.float32)
    ts_ref[...] = t * dis


_tc_c = pl.pallas_call(
    _tcC_body,
    grid=(GRID,),
    in_specs=[
        pl.BlockSpec((R, C), lambda i: (i, 0)),
        pl.BlockSpec((R, C), lambda i: (i, 0)),
        pl.BlockSpec((R, 1), lambda i: (i, 0)),
        pl.BlockSpec((C, H), lambda i: (0, 0)),
        pl.BlockSpec((C, H), lambda i: (0, 0)),
        pl.BlockSpec((1, H), lambda i: (0, 0)),
        pl.BlockSpec((H, C), lambda i: (0, 0)),
    ],
    out_specs=pl.BlockSpec((R, C), lambda i: (i, 0)),
    out_shape=jax.ShapeDtypeStruct((N, C), jnp.float32),
)


def _tcE_body(a0_ref, a1_ref, dis_ref, b2_ref, out_ref):
    out_ref[...] = ((a0_ref[...] + a1_ref[...]) * dis_ref[...]
                    + b2_ref[...])


_tc_e = pl.pallas_call(
    _tcE_body,
    grid=(GRID,),
    in_specs=[
        pl.BlockSpec((R, C), lambda i: (i, 0)),
        pl.BlockSpec((R, C), lambda i: (i, 0)),
        pl.BlockSpec((R, 1), lambda i: (i, 0)),
        pl.BlockSpec((1, C), lambda i: (0, 0)),
    ],
    out_specs=pl.BlockSpec((R, C), lambda i: (i, 0)),
    out_shape=jax.ShapeDtypeStruct((N, C), jnp.float32),
)


def kernel(x, edge_index, W1, b1, W2, b2):
    ei = edge_index.astype(jnp.int32)
    src, dst = ei[0], ei[1]

    pad = jnp.full((EPAD - E,), N, jnp.int32)
    src_d = jnp.concatenate([src, pad]).reshape(ECH, B)
    dst_d = jnp.concatenate([dst, pad]).reshape(ECH, B)

    srcs_b = jnp.concatenate([src, pad, src + T, pad + T]).reshape(2 * ECH, B)

    ones128 = jnp.ones((B, C), jnp.float32)
    zbig = jnp.zeros((T, C), jnp.float32)
    zpad = jnp.zeros((T - N, C), jnp.float32)

    degacc = _sc_deg(dst_d, ones128, zbig)
    xs0, xs1, dis = _tc_a(degacc[0], degacc[1], x)
    xs = jnp.concatenate([xs0, zpad, xs1, zpad], axis=0)
    acc1 = _sc_aggB(xs, srcs_b, dst_d)
    ts = _tc_c(acc1[0], acc1[1], dis, W1[:C], W1[C:], b1.reshape(1, H), W2)
    ts_pad = jnp.concatenate([ts, zpad], axis=0)
    acc2 = _sc_aggD(ts_pad, src_d, dst_d, zbig)
    return _tc_e(acc2[0], acc2[1], dis, b2.reshape(1, C))
